# Initial kernel scaffold; baseline (speedup 1.0000x reference)
import jax
import jax.numpy as jnp
from jax.experimental import pallas as pl


def kernel(x, task_id, emb0, emb1, emb2, mlp0_W1, mlp0_b1, mlp0_W2, mlp0_b2, mlp1_W1, mlp1_b1, mlp1_W2, mlp1_b2):
    raise NotImplementedError("write your pallas kernel here")


# speedup vs baseline: 3.7999x; 3.7999x over previous
import functools

import jax
import jax.numpy as jnp
from jax import lax
from jax.experimental import pallas as pl

BATCH = 16384
D0 = 4096
M = 64
H = 64
BB = 512


def _top2_onehot(e, n):
    it = lax.broadcasted_iota(jnp.int32, e.shape, 0)
    m1 = jnp.max(e, axis=0, keepdims=True)
    i1 = jnp.min(jnp.where(e == m1, it, n), axis=0, keepdims=True)
    em = jnp.where(it == i1, -jnp.inf, e)
    m2 = jnp.max(em, axis=0, keepdims=True)
    i2 = jnp.min(jnp.where(em == m2, it, n), axis=0, keepdims=True)
    s1 = (it == i1).astype(jnp.float32)
    s2 = (it == i2).astype(jnp.float32)
    return jnp.concatenate([s1, s2], axis=1)


def _route_body(e0_ref, e1_ref, e2_ref, s0_ref, s1_ref, s2_ref):
    s0_ref[...] = _top2_onehot(e0_ref[...], D0)
    s1_ref[...] = _top2_onehot(e1_ref[...], M)
    s2_ref[...] = _top2_onehot(e2_ref[...], M)


def _main_body(x_ref, s0_ref, p0_ref, b1f0_ref, r0_ref, b2f0_ref,
               s1_ref, p1_ref, b1f1_ref, r1_ref, b2f1_ref, s2_ref, o_ref):
    f32 = jnp.float32
    u0 = jnp.dot(x_ref[...], s0_ref[...], preferred_element_type=f32)
    h0 = jnp.maximum(jnp.dot(u0, p0_ref[...], preferred_element_type=f32)
                     + b1f0_ref[0:1, :], 0.0)
    x1 = jnp.dot(h0, r0_ref[...], preferred_element_type=f32) + b2f0_ref[0:1, :]
    u1 = jnp.dot(x1, s1_ref[...], preferred_element_type=f32)
    h1 = jnp.maximum(jnp.dot(u1, p1_ref[...], preferred_element_type=f32)
                     + b1f1_ref[0:1, :], 0.0)
    x2 = jnp.dot(h1, r1_ref[...], preferred_element_type=f32) + b2f1_ref[0:1, :]
    o_ref[...] = jax.nn.sigmoid(
        jnp.dot(x2, s2_ref[...], preferred_element_type=f32))


def _block_weights(W1, b1, W2, b2):
    eye = jnp.eye(M, dtype=jnp.float32)
    P = jnp.einsum('mn,mhi->imnh', eye, W1).reshape(2 * M, M * H)
    b1f = jnp.broadcast_to(b1.reshape(1, M * H), (8, M * H))
    R = jnp.einsum('mn,mh->mhn', eye, W2[:, 0, :]).reshape(M * H, M)
    b2f = jnp.broadcast_to(b2.reshape(1, M), (8, M))
    return P, b1f, R, b2f


def kernel(x, task_id, emb0, emb1, emb2,
           mlp0_W1, mlp0_b1, mlp0_W2, mlp0_b2,
           mlp1_W1, mlp1_b1, mlp1_W2, mlp1_b2):
    e0 = lax.dynamic_index_in_dim(emb0, task_id, 0, keepdims=False)
    e1 = lax.dynamic_index_in_dim(emb1, task_id, 0, keepdims=False)
    e2 = lax.dynamic_index_in_dim(emb2, task_id, 0, keepdims=False)

    s0, s1, s2 = pl.pallas_call(
        _route_body,
        out_shape=(
            jax.ShapeDtypeStruct((D0, 2 * M), jnp.float32),
            jax.ShapeDtypeStruct((M, 2 * M), jnp.float32),
            jax.ShapeDtypeStruct((M, 2), jnp.float32),
        ),
    )(e0, e1, e2)

    P0, b1f0, R0, b2f0 = _block_weights(mlp0_W1, mlp0_b1, mlp0_W2, mlp0_b2)
    P1, b1f1, R1, b2f1 = _block_weights(mlp1_W1, mlp1_b1, mlp1_W2, mlp1_b2)

    nblk = BATCH // BB
    full = lambda shape: pl.BlockSpec(shape, lambda i: (0, 0))
    out = pl.pallas_call(
        _main_body,
        grid=(nblk,),
        in_specs=[
            pl.BlockSpec((BB, D0), lambda i: (i, 0)),
            full((D0, 2 * M)),
            full((2 * M, M * H)), full((8, M * H)), full((M * H, M)), full((8, M)),
            full((M, 2 * M)),
            full((2 * M, M * H)), full((8, M * H)), full((M * H, M)), full((8, M)),
            full((M, 2)),
        ],
        out_specs=pl.BlockSpec((BB, 2), lambda i: (i, 0)),
        out_shape=jax.ShapeDtypeStruct((BATCH, 2), jnp.float32),
    )(x, s0, P0, b1f0, R0, b2f0, s1, P1, b1f1, R1, b2f1, s2)
    return out
